# Initial kernel scaffold; baseline (speedup 1.0000x reference)
import jax
import jax.numpy as jnp
from jax.experimental import pallas as pl


def kernel(u, v, e_indices, e_values, Wg, bg, Wf, bf):
    raise NotImplementedError("write your pallas kernel here")


# speedup vs baseline: 3.5253x; 3.5253x over previous
import functools

import jax
import jax.numpy as jnp
from jax import lax
from jax.experimental import pallas as pl
from jax.experimental.pallas import tpu as pltpu
from jax.experimental.pallas import tpu_sc as plsc

_U = 10000
_E = 320000
_DG = 64
_C = 256
_NCH = _E // _C
_NW = 32
_IDXK = _C // 128
_ROWS_PER_SUB = 624
_TAIL_ROW0 = 16 * _ROWS_PER_SUB
_TAIL_ROWS = _U - _TAIL_ROW0


def _mm_body(x_ref, w_ref, b_ref, o_ref):
    o_ref[...] = (
        jnp.dot(x_ref[...], w_ref[...], preferred_element_type=jnp.float32)
        + b_ref[...]
    )


def _mm(x, w, b, br):
    m, k = x.shape
    n = w.shape[1]
    return pl.pallas_call(
        _mm_body,
        grid=(m // br,),
        in_specs=[
            pl.BlockSpec((br, k), lambda i: (i, 0)),
            pl.BlockSpec((k, n), lambda i: (0, 0)),
            pl.BlockSpec((1, n), lambda i: (0, 0)),
        ],
        out_specs=pl.BlockSpec((br, n), lambda i: (i, 0)),
        out_shape=jax.ShapeDtypeStruct((m, n), jnp.float32),
    )(x, w, b.reshape(1, n))


def _f_body(u_ref, a0_ref, a1_ref, wu_ref, wa_ref, b_ref, o_ref):
    acc = jnp.dot(u_ref[...], wu_ref[...], preferred_element_type=jnp.float32)
    acc = acc + jnp.dot(
        a0_ref[...] + a1_ref[...], wa_ref[...],
        preferred_element_type=jnp.float32,
    )
    o_ref[...] = jnp.maximum(acc + b_ref[...], 0.0)


def _f_mm(u, a0, a1, wu, wa, b, br):
    m, k = u.shape
    ka = a0.shape[1]
    n = wu.shape[1]
    return pl.pallas_call(
        _f_body,
        grid=(m // br,),
        in_specs=[
            pl.BlockSpec((br, k), lambda i: (i, 0)),
            pl.BlockSpec((br, ka), lambda i: (i, 0)),
            pl.BlockSpec((br, ka), lambda i: (i, 0)),
            pl.BlockSpec((k, n), lambda i: (0, 0)),
            pl.BlockSpec((ka, n), lambda i: (0, 0)),
            pl.BlockSpec((1, n), lambda i: (0, 0)),
        ],
        out_specs=pl.BlockSpec((br, n), lambda i: (i, 0)),
        out_shape=jax.ShapeDtypeStruct((m, n), jnp.float32),
    )(u, a0, a1, wu, wa, b.reshape(1, n))


def _sc_edge_body(ug, vg, eg3, dst3, src3, zeros_hbm, out, idx_d, idx_s, acc,
                  bu, bv, agg_sh, sem):
    cid = lax.axis_index("c")
    sid = lax.axis_index("s")
    wid = sid * 2 + cid

    row0 = sid * _ROWS_PER_SUB
    pltpu.sync_copy(
        zeros_hbm.at[pl.ds(row0, _ROWS_PER_SUB)],
        agg_sh.at[pl.ds(row0, _ROWS_PER_SUB)],
    )

    @pl.when(sid == 15)
    def _():
        pltpu.sync_copy(
            zeros_hbm.at[pl.ds(_TAIL_ROW0, _TAIL_ROWS)],
            agg_sh.at[pl.ds(_TAIL_ROW0, _TAIL_ROWS)],
        )

    plsc.subcore_barrier()

    def chunk_body(k, carry):
        g = wid + k * _NW

        @pl.when(g < _NCH)
        def _():
            pltpu.sync_copy(dst3.at[g], idx_d)
            pltpu.sync_copy(src3.at[g], idx_s)
            cps = [pltpu.make_async_copy(eg3.at[g], acc, sem)]
            for j in range(_IDXK):
                cps.append(pltpu.make_async_copy(
                    ug.at[idx_d.at[j]], bu.at[pl.ds(j * 128, 128)], sem))
                cps.append(pltpu.make_async_copy(
                    vg.at[idx_s.at[j]], bv.at[pl.ds(j * 128, 128)], sem))
            for cp in cps:
                cp.start()
            for cp in cps:
                cp.wait()

            def row_body(i, c2):
                for c in range(_DG // 16):
                    sl = pl.ds(c * 16, 16)
                    s = acc[i, sl] + bu[i, sl] + bv[i, sl]
                    acc[i, sl] = jnp.maximum(s, 0.0)
                return c2

            lax.fori_loop(0, _C, row_body, 0)

            for j in range(_IDXK):
                pltpu.sync_copy(
                    acc.at[pl.ds(j * 128, 128)],
                    agg_sh.at[idx_d.at[j]],
                    add=True,
                )

        return carry

    n_iter = (_NCH + _NW - 1) // _NW
    lax.fori_loop(0, n_iter, chunk_body, 0)

    plsc.subcore_barrier()
    pltpu.sync_copy(
        agg_sh.at[pl.ds(row0, _ROWS_PER_SUB)],
        out.at[cid, pl.ds(row0, _ROWS_PER_SUB)],
    )

    @pl.when(sid == 15)
    def _():
        pltpu.sync_copy(
            agg_sh.at[pl.ds(_TAIL_ROW0, _TAIL_ROWS)],
            out.at[cid, pl.ds(_TAIL_ROW0, _TAIL_ROWS)],
        )


@functools.cache
def _get_sc_edge():
    mesh = plsc.VectorSubcoreMesh(
        core_axis_name="c", subcore_axis_name="s", num_cores=2,
        num_subcores=16,
    )
    return pl.kernel(
        _sc_edge_body,
        out_type=jax.ShapeDtypeStruct((2, _U, _DG), jnp.float32),
        mesh=mesh,
        scratch_types=[
            pltpu.VMEM((_IDXK, 128), jnp.int32),
            pltpu.VMEM((_IDXK, 128), jnp.int32),
            pltpu.VMEM((_C, _DG), jnp.float32),
            pltpu.VMEM((_C, _DG), jnp.float32),
            pltpu.VMEM((_C, _DG), jnp.float32),
            pltpu.VMEM_SHARED((_U, _DG), jnp.float32),
            pltpu.SemaphoreType.DMA,
        ],
        compiler_params=pltpu.CompilerParams(use_tc_tiling_on_sc=False),
    )


@jax.jit
def _impl(u, v, e_indices, e_values, Wg, bg, Wf, bf):
    f_dim = u.shape[1]
    g_dim = v.shape[1]
    src = e_indices[0].astype(jnp.int32)
    dst = e_indices[1].astype(jnp.int32)

    ug_t = _mm(u, Wg[:f_dim], bg, 1000)
    vg_t = _mm(v, Wg[f_dim:f_dim + g_dim], jnp.zeros((_DG,), jnp.float32),
               1000)
    eg_t = _mm(e_values, Wg[f_dim + g_dim:],
               jnp.zeros((_DG,), jnp.float32), 1000)

    dst3 = dst.reshape(_NCH, _IDXK, 128)
    src3 = src.reshape(_NCH, _IDXK, 128)
    eg3 = eg_t.reshape(_NCH, _C, _DG)
    zeros = jnp.zeros((_U, _DG), jnp.float32)

    agg2 = _get_sc_edge()(ug_t, vg_t, eg3, dst3, src3, zeros)

    return _f_mm(u, agg2[0], agg2[1], Wf[:f_dim], Wf[f_dim:], bf, 1000)


def kernel(u, v, e_indices, e_values, Wg, bg, Wf, bf):
    return _impl(u, v, e_indices, e_values, Wg, bg, Wf, bf)


# speedup vs baseline: 4.3561x; 1.2357x over previous
import functools

import jax
import jax.numpy as jnp
from jax import lax
from jax.experimental import pallas as pl
from jax.experimental.pallas import tpu as pltpu
from jax.experimental.pallas import tpu_sc as plsc

_U = 10000
_E = 320000
_DG = 64
_C = 256
_NCH = _E // _C
_NW = 32
_IDXK = _C // 128
_ROWS_PER_SUB = 624
_TAIL_ROW0 = 16 * _ROWS_PER_SUB
_TAIL_ROWS = _U - _TAIL_ROW0


def _mm_body(x_ref, w_ref, b_ref, o_ref):
    o_ref[...] = (
        jnp.dot(x_ref[...], w_ref[...], preferred_element_type=jnp.float32)
        + b_ref[...]
    )


def _mm(x, w, b, br):
    m, k = x.shape
    n = w.shape[1]
    return pl.pallas_call(
        _mm_body,
        grid=(m // br,),
        in_specs=[
            pl.BlockSpec((br, k), lambda i: (i, 0)),
            pl.BlockSpec((k, n), lambda i: (0, 0)),
            pl.BlockSpec((1, n), lambda i: (0, 0)),
        ],
        out_specs=pl.BlockSpec((br, n), lambda i: (i, 0)),
        out_shape=jax.ShapeDtypeStruct((m, n), jnp.float32),
    )(x, w, b.reshape(1, n))


def _f_body(u_ref, a0_ref, a1_ref, wu_ref, wa_ref, b_ref, o_ref):
    acc = jnp.dot(u_ref[...], wu_ref[...], preferred_element_type=jnp.float32)
    acc = acc + jnp.dot(
        a0_ref[...] + a1_ref[...], wa_ref[...],
        preferred_element_type=jnp.float32,
    )
    o_ref[...] = jnp.maximum(acc + b_ref[...], 0.0)


def _f_mm(u, a0, a1, wu, wa, b, br):
    m, k = u.shape
    ka = a0.shape[1]
    n = wu.shape[1]
    return pl.pallas_call(
        _f_body,
        grid=(m // br,),
        in_specs=[
            pl.BlockSpec((br, k), lambda i: (i, 0)),
            pl.BlockSpec((br, ka), lambda i: (i, 0)),
            pl.BlockSpec((br, ka), lambda i: (i, 0)),
            pl.BlockSpec((k, n), lambda i: (0, 0)),
            pl.BlockSpec((ka, n), lambda i: (0, 0)),
            pl.BlockSpec((1, n), lambda i: (0, 0)),
        ],
        out_specs=pl.BlockSpec((br, n), lambda i: (i, 0)),
        out_shape=jax.ShapeDtypeStruct((m, n), jnp.float32),
    )(u, a0, a1, wu, wa, b.reshape(1, n))


def _sc_edge_body(ug, vg, eg3, dst3, src3, zeros_hbm, out, idx_d, idx_s, acc,
                  bu, bv, agg_sh, sem):
    cid = lax.axis_index("c")
    sid = lax.axis_index("s")
    wid = sid * 2 + cid

    row0 = sid * _ROWS_PER_SUB
    pltpu.sync_copy(
        zeros_hbm.at[pl.ds(row0, _ROWS_PER_SUB)],
        agg_sh.at[pl.ds(row0, _ROWS_PER_SUB)],
    )

    @pl.when(sid == 15)
    def _():
        pltpu.sync_copy(
            zeros_hbm.at[pl.ds(_TAIL_ROW0, _TAIL_ROWS)],
            agg_sh.at[pl.ds(_TAIL_ROW0, _TAIL_ROWS)],
        )

    plsc.subcore_barrier()

    def chunk_body(k, carry):
        g = wid + k * _NW

        @pl.when(g < _NCH)
        def _():
            pltpu.sync_copy(dst3.at[g], idx_d)
            pltpu.sync_copy(src3.at[g], idx_s)
            cps = [pltpu.make_async_copy(eg3.at[g], acc, sem)]
            for j in range(_IDXK):
                cps.append(pltpu.make_async_copy(
                    ug.at[idx_d.at[j]], bu.at[pl.ds(j * 128, 128)], sem))
                cps.append(pltpu.make_async_copy(
                    vg.at[idx_s.at[j]], bv.at[pl.ds(j * 128, 128)], sem))
            for cp in cps:
                cp.start()
            for cp in cps:
                cp.wait()

            def row_body(i, c2):
                for c in range(_DG // 16):
                    sl = pl.ds(c * 16, 16)
                    s = acc[i, sl] + bu[i, sl] + bv[i, sl]
                    acc[i, sl] = jnp.maximum(s, 0.0)
                return c2

            lax.fori_loop(0, _C, row_body, 0)

            for j in range(_IDXK):
                pltpu.sync_copy(
                    acc.at[pl.ds(j * 128, 128)],
                    agg_sh.at[idx_d.at[j]],
                    add=True,
                )

        return carry

    n_iter = (_NCH + _NW - 1) // _NW
    lax.fori_loop(0, n_iter, chunk_body, 0)

    plsc.subcore_barrier()
    pltpu.sync_copy(
        agg_sh.at[pl.ds(row0, _ROWS_PER_SUB)],
        out.at[cid, pl.ds(row0, _ROWS_PER_SUB)],
    )

    @pl.when(sid == 15)
    def _():
        pltpu.sync_copy(
            agg_sh.at[pl.ds(_TAIL_ROW0, _TAIL_ROWS)],
            out.at[cid, pl.ds(_TAIL_ROW0, _TAIL_ROWS)],
        )


@functools.cache
def _get_sc_edge():
    mesh = plsc.VectorSubcoreMesh(
        core_axis_name="c", subcore_axis_name="s", num_cores=2,
        num_subcores=16,
    )
    return pl.kernel(
        _sc_edge_body,
        out_type=jax.ShapeDtypeStruct((2, _U, _DG), jnp.float32),
        mesh=mesh,
        scratch_types=[
            pltpu.VMEM((_IDXK, 128), jnp.int32),
            pltpu.VMEM((_IDXK, 128), jnp.int32),
            pltpu.VMEM((_C, _DG), jnp.float32),
            pltpu.VMEM((_C, _DG), jnp.float32),
            pltpu.VMEM((_C, _DG), jnp.float32),
            pltpu.VMEM_SHARED((_U, _DG), jnp.float32),
            pltpu.SemaphoreType.DMA,
        ],
        compiler_params=pltpu.CompilerParams(use_tc_tiling_on_sc=False),
    )


@jax.jit
def _impl(u, v, e_indices, e_values, Wg, bg, Wf, bf):
    f_dim = u.shape[1]
    g_dim = v.shape[1]
    src = e_indices[0].astype(jnp.int32)
    dst = e_indices[1].astype(jnp.int32)

    ug_t = _mm(u, Wg[:f_dim], bg, 1000)
    vg_t = _mm(v, Wg[f_dim:f_dim + g_dim], jnp.zeros((_DG,), jnp.float32),
               1000)
    eg_t = _mm(e_values, Wg[f_dim + g_dim:],
               jnp.zeros((_DG,), jnp.float32), 8000)

    dst3 = dst.reshape(_NCH, _IDXK, 128)
    src3 = src.reshape(_NCH, _IDXK, 128)
    eg3 = eg_t.reshape(_NCH, _C, _DG)
    zeros = jnp.zeros((_U, _DG), jnp.float32)

    agg2 = _get_sc_edge()(ug_t, vg_t, eg3, dst3, src3, zeros)

    return _f_mm(u, agg2[0], agg2[1], Wf[:f_dim], Wf[f_dim:], bf, 1000)


def kernel(u, v, e_indices, e_values, Wg, bg, Wf, bf):
    return _impl(u, v, e_indices, e_values, Wg, bg, Wf, bf)
